# baseline (device time: 147893 ns/iter reference)
import jax
import jax.numpy as jnp
from jax import lax
from jax.experimental import pallas as pl
from jax.experimental.pallas import tpu as pltpu

SQ = 2048
D = 1024
HQ = 8
DH = 128
BLK = 64
SCALE = 0.08838834764831843
TILES = [(0, 256), (256, 256), (512, 256), (768, 256), (1024, 256),
         (1280, 256), (1536, 256), (1792, 128), (1920, 128)]
N_TILES = len(TILES)
N_CHAIN = 7
NT = (((1,), (1,)), ((), ()))
NN = (((1,), (0,)), ((), ()))
MESH = pl.DeviceIdType.MESH


def kernel(x, Wq, K_ext, V_ext, Wo):
    bf = jnp.bfloat16
    f32 = jnp.float32
    x2 = x.reshape(SQ, D).astype(bf)
    K2 = K_ext.reshape(SQ, D).astype(bf)
    V2 = V_ext.reshape(SQ, D).astype(bf)
    Wqb = Wq.astype(bf)
    Wob = Wo.astype(bf)

    def body(x_ref, wq_ref, k_ref, v_ref, wo_ref, out_ref,
             qbf, vaug, ctxbuf, send_sems, recv_sems, fwd_sems):
        my = lax.axis_index("i")

        def tile_ref(t):
            r0, rows = TILES[t]
            return ctxbuf.at[r0:r0 + rows, :]

        barrier = pltpu.get_barrier_semaphore()

        @pl.when((my == 1) | (my == 3))
        def _():
            for d in (0, 2):
                pl.semaphore_signal(barrier, inc=1, device_id=(d,),
                                    device_id_type=MESH)
            pl.semaphore_wait(barrier, 1)

        @pl.when(my == 2)
        def _():
            for d in (0, 1, 3):
                pl.semaphore_signal(barrier, inc=1, device_id=(d,),
                                    device_id_type=MESH)
            pl.semaphore_wait(barrier, 2)

        @pl.when(my == 0)
        def _():
            pl.semaphore_wait(barrier, 3)

        @pl.when(my == 0)
        def _producer():
            qbf[...] = (jnp.dot(x_ref[...], wq_ref[...],
                                preferred_element_type=f32)
                        * SCALE).astype(bf)
            ones = jnp.ones((SQ, DH), bf)
            for h in range(HQ):
                vaug[:, h * 2 * DH:h * 2 * DH + DH] = \
                    v_ref[:, h * DH:(h + 1) * DH]
                vaug[:, h * 2 * DH + DH:(h + 1) * 2 * DH] = ones
            biases = {}
            for rows in {256, 128}:
                r = lax.broadcasted_iota(jnp.int32, (rows, rows), 0) // BLK
                c = lax.broadcasted_iota(jnp.int32, (rows, rows), 1) // BLK
                biases[rows] = jnp.where(r >= c, f32(0), f32(-1e9)).astype(bf)

            rdmas = []
            for t in range(N_TILES):
                r0, rows = TILES[t]
                for h in range(HQ):
                    hs = slice(h * DH, (h + 1) * DH)
                    vs = slice(h * 2 * DH, (h + 1) * 2 * DH)
                    qt = qbf[r0:r0 + rows, hs]
                    sd = lax.dot_general(
                        qt, k_ref[r0:r0 + rows, hs], NT,
                        preferred_element_type=f32)
                    wd = jnp.exp(sd.astype(bf) + biases[rows])
                    res = lax.dot_general(
                        wd, vaug[r0:r0 + rows, vs], NN,
                        preferred_element_type=f32)
                    if r0 > 0:
                        s1 = lax.dot_general(
                            qt, k_ref[0:r0, hs], NT,
                            preferred_element_type=f32)
                        w1 = jnp.exp(s1.astype(bf))
                        res = res + lax.dot_general(
                            w1, vaug[0:r0, vs], NN,
                            preferred_element_type=f32)
                    ctxbuf[r0:r0 + rows, hs] = \
                        (res[:, 0:DH] / res[:, DH:DH + 1]).astype(bf)
                dests = ((1,) if t % 2 == 0 else (3,)) if t < N_CHAIN \
                    else (1, 2, 3)
                for i, d in enumerate(dests):
                    rd = pltpu.make_async_remote_copy(
                        src_ref=tile_ref(t), dst_ref=tile_ref(t),
                        send_sem=send_sems.at[t, i], recv_sem=recv_sems.at[t],
                        device_id=(d,), device_id_type=MESH)
                    rd.start()
                    rdmas.append(rd)
            out_ref[...] = jnp.dot(ctxbuf[...], wo_ref[...],
                                   preferred_element_type=f32).astype(bf)
            for rd in rdmas:
                rd.wait_send()

        @pl.when(my != 0)
        def _consumer():
            fwds = []
            for t in range(N_TILES):
                r0, rows = TILES[t]
                rd = pltpu.make_async_remote_copy(
                    src_ref=tile_ref(t), dst_ref=tile_ref(t),
                    send_sem=send_sems.at[t, 0], recv_sem=recv_sems.at[t],
                    device_id=(0,), device_id_type=MESH)
                rd.wait_recv()
                if t < N_CHAIN:
                    if t % 2 == 0:
                        cond = (my == 1) | (my == 2)
                        target = jnp.where(my == 1, 2, 3)
                    else:
                        cond = (my == 3) | (my == 2)
                        target = jnp.where(my == 3, 2, 1)
                    fwds.append((t, cond, target))

                    @pl.when(cond)
                    def _fwd(t=t, target=target):
                        f = pltpu.make_async_remote_copy(
                            src_ref=tile_ref(t), dst_ref=tile_ref(t),
                            send_sem=fwd_sems.at[t], recv_sem=recv_sems.at[t],
                            device_id=(target,), device_id_type=MESH)
                        f.start()

                out_ref[r0:r0 + rows, :] = jnp.dot(
                    ctxbuf[r0:r0 + rows, :], wo_ref[...],
                    preferred_element_type=f32).astype(bf)

            for t, cond, target in fwds:
                @pl.when(cond)
                def _fwd_wait(t=t, target=target):
                    f = pltpu.make_async_remote_copy(
                        src_ref=tile_ref(t), dst_ref=tile_ref(t),
                        send_sem=fwd_sems.at[t], recv_sem=recv_sems.at[t],
                        device_id=(target,), device_id_type=MESH)
                    f.wait_send()

    out = pl.pallas_call(
        body,
        out_shape=jax.ShapeDtypeStruct((SQ, D), bf),
        in_specs=[pl.BlockSpec(memory_space=pltpu.VMEM)] * 5,
        out_specs=pl.BlockSpec(memory_space=pltpu.VMEM),
        scratch_shapes=[
            pltpu.VMEM((SQ, D), bf),
            pltpu.VMEM((SQ, 2 * D), bf),
            pltpu.VMEM((SQ, D), bf),
            pltpu.SemaphoreType.DMA((N_TILES, 3)),
            pltpu.SemaphoreType.DMA((N_TILES,)),
            pltpu.SemaphoreType.DMA((N_TILES,)),
        ],
        compiler_params=pltpu.CompilerParams(collective_id=0),
    )(x2, Wqb, K2, V2, Wob)

    return out.reshape(1, SQ, D)


# device time: 73019 ns/iter; 2.0254x vs baseline; 2.0254x over previous
import jax
import jax.numpy as jnp
from jax import lax
from jax.experimental import pallas as pl
from jax.experimental.pallas import tpu as pltpu

SQ = 2048
D = 1024
HQ = 8
DH = 128
BLK = 64
SCALE = 0.08838834764831843
TILES = [(0, 256), (256, 256), (512, 256), (768, 256), (1024, 256),
         (1280, 256), (1536, 256), (1792, 128), (1920, 128)]
N_TILES = len(TILES)
N_CHAIN = 7
NT = (((1,), (1,)), ((), ()))
NN = (((1,), (0,)), ((), ()))
MESH = pl.DeviceIdType.MESH


def kernel(x, Wq, K_ext, V_ext, Wo):
    bf = jnp.bfloat16
    f32 = jnp.float32
    x2 = x.reshape(SQ, D).astype(bf)
    K2 = K_ext.reshape(SQ, D).astype(bf)
    V2 = V_ext.reshape(SQ, D).astype(bf)
    Wqb = Wq.astype(bf)
    Wob = Wo.astype(bf)

    def body(x_ref, wq_ref, k_ref, v_ref, wo_ref, out_ref,
             qbf, vaug, ctxbuf, send_sems, recv_sems, fwd_sems):
        my = lax.axis_index("i")

        def tile_ref(t):
            r0, rows = TILES[t]
            return ctxbuf.at[r0:r0 + rows, :]

        barrier = pltpu.get_barrier_semaphore()

        @pl.when((my == 1) | (my == 3))
        def _():
            for d in (0, 2):
                pl.semaphore_signal(barrier, inc=1, device_id=(d,),
                                    device_id_type=MESH)
            pl.semaphore_wait(barrier, 1)

        @pl.when(my == 2)
        def _():
            for d in (0, 1, 3):
                pl.semaphore_signal(barrier, inc=1, device_id=(d,),
                                    device_id_type=MESH)
            pl.semaphore_wait(barrier, 2)

        @pl.when(my == 0)
        def _():
            pl.semaphore_wait(barrier, 3)

        @pl.when(my == 0)
        def _producer():
            qbf[...] = (jnp.dot(x_ref[...], wq_ref[...],
                                preferred_element_type=f32)
                        * SCALE).astype(bf)
            ones = jnp.ones((SQ, DH), bf)
            for h in range(HQ):
                vaug[:, h * 2 * DH:h * 2 * DH + DH] = \
                    v_ref[:, h * DH:(h + 1) * DH]
                vaug[:, h * 2 * DH + DH:(h + 1) * 2 * DH] = ones
            biases = {}
            for rows in {256, 128}:
                r = lax.broadcasted_iota(jnp.int32, (rows, rows), 0) // BLK
                c = lax.broadcasted_iota(jnp.int32, (rows, rows), 1) // BLK
                biases[rows] = jnp.where(r >= c, f32(0), f32(-1e9)).astype(bf)

            rdmas = []
            for t in range(N_TILES):
                r0, rows = TILES[t]
                for h in range(HQ):
                    hs = slice(h * DH, (h + 1) * DH)
                    vs = slice(h * 2 * DH, (h + 1) * 2 * DH)
                    qt = qbf[r0:r0 + rows, hs]
                    sd = lax.dot_general(
                        qt, k_ref[r0:r0 + rows, hs], NT,
                        preferred_element_type=f32)
                    wd = jnp.exp(sd.astype(bf) + biases[rows])
                    res = lax.dot_general(
                        wd, vaug[r0:r0 + rows, vs], NN,
                        preferred_element_type=f32)
                    if r0 > 0:
                        s1 = lax.dot_general(
                            qt, k_ref[0:r0, hs], NT,
                            preferred_element_type=f32)
                        w1 = jnp.exp(s1.astype(bf))
                        res = res + lax.dot_general(
                            w1, vaug[0:r0, vs], NN,
                            preferred_element_type=f32)
                    ctxbuf[r0:r0 + rows, hs] = \
                        (res[:, 0:DH] / res[:, DH:DH + 1]).astype(bf)
                dests = ((1,) if t % 2 == 0 else (3,)) if t < N_CHAIN \
                    else (1, 2, 3)
                for i, d in enumerate(dests):
                    rd = pltpu.make_async_remote_copy(
                        src_ref=tile_ref(t), dst_ref=tile_ref(t),
                        send_sem=send_sems.at[t, i], recv_sem=recv_sems.at[t],
                        device_id=(d,), device_id_type=MESH)
                    rd.start()
                    rdmas.append(rd)
            out_ref[...] = jnp.dot(ctxbuf[...], wo_ref[...],
                                   preferred_element_type=f32).astype(bf)
            for rd in rdmas:
                rd.wait_send()

        @pl.when(my != 0)
        def _consumer():
            def recv(t):
                rd = pltpu.make_async_remote_copy(
                    src_ref=tile_ref(t), dst_ref=tile_ref(t),
                    send_sem=send_sems.at[t, 0], recv_sem=recv_sems.at[t],
                    device_id=(0,), device_id_type=MESH)
                rd.wait_recv()

            def fwd_descriptor(t, target):
                return pltpu.make_async_remote_copy(
                    src_ref=tile_ref(t), dst_ref=tile_ref(t),
                    send_sem=fwd_sems.at[t], recv_sem=recv_sems.at[t],
                    device_id=(target,), device_id_type=MESH)

            evens = [t for t in range(N_CHAIN) if t % 2 == 0]
            odds = [t for t in range(N_CHAIN) if t % 2 == 1]
            fwds = []

            for mine, tiles in (((my == 1), evens), ((my == 3), odds)):
                @pl.when(mine)
                def _first_hop(tiles=tiles):
                    for t in tiles:
                        recv(t)
                        fwd_descriptor(t, 2).start()
                for t in tiles:
                    fwds.append((t, mine, 2))

            @pl.when(my == 2)
            def _mid_hop():
                for t in range(N_CHAIN):
                    recv(t)
                    fwd_descriptor(t, 3 if t % 2 == 0 else 1).start()
            for t in range(N_CHAIN):
                fwds.append((t, my == 2, 3 if t % 2 == 0 else 1))

            @pl.when(my == 1)
            def _last_hop_odds():
                for t in odds:
                    recv(t)

            @pl.when(my == 3)
            def _last_hop_evens():
                for t in evens:
                    recv(t)

            for t in range(N_CHAIN, N_TILES):
                recv(t)

            out_ref[...] = jnp.dot(ctxbuf[...], wo_ref[...],
                                   preferred_element_type=f32).astype(bf)

            for t, cond, target in fwds:
                @pl.when(cond)
                def _fwd_wait(t=t, target=target):
                    fwd_descriptor(t, target).wait_send()

    out = pl.pallas_call(
        body,
        out_shape=jax.ShapeDtypeStruct((SQ, D), bf),
        in_specs=[pl.BlockSpec(memory_space=pltpu.VMEM)] * 5,
        out_specs=pl.BlockSpec(memory_space=pltpu.VMEM),
        scratch_shapes=[
            pltpu.VMEM((SQ, D), bf),
            pltpu.VMEM((SQ, 2 * D), bf),
            pltpu.VMEM((SQ, D), bf),
            pltpu.SemaphoreType.DMA((N_TILES, 3)),
            pltpu.SemaphoreType.DMA((N_TILES,)),
            pltpu.SemaphoreType.DMA((N_TILES,)),
        ],
        compiler_params=pltpu.CompilerParams(collective_id=0),
    )(x2, Wqb, K2, V2, Wob)

    return out.reshape(1, SQ, D)


# device time: 70664 ns/iter; 2.0929x vs baseline; 1.0333x over previous
import jax
import jax.numpy as jnp
from jax import lax
from jax.experimental import pallas as pl
from jax.experimental.pallas import tpu as pltpu

SQ = 2048
D = 1024
HQ = 8
DH = 128
BLK = 64
SCALE = 0.08838834764831843
TILES = [(0, 256), (256, 256), (512, 256), (768, 256), (1024, 256),
         (1280, 256), (1536, 256), (1792, 128), (1920, 128)]
N_TILES = len(TILES)
N_CHAIN = 5
NT = (((1,), (1,)), ((), ()))
NN = (((1,), (0,)), ((), ()))
MESH = pl.DeviceIdType.MESH


def kernel(x, Wq, K_ext, V_ext, Wo):
    bf = jnp.bfloat16
    f32 = jnp.float32
    x2 = x.reshape(SQ, D).astype(bf)
    K2 = K_ext.reshape(SQ, D).astype(bf)
    V2 = V_ext.reshape(SQ, D).astype(bf)
    Wqb = Wq.astype(bf)
    Wob = Wo.astype(bf)

    def body(x_ref, wq_ref, k_ref, v_ref, wo_ref, out_ref,
             qbf, vaug, ctxbuf, send_sems, recv_sems, fwd_sems):
        my = lax.axis_index("i")

        def tile_ref(t):
            r0, rows = TILES[t]
            return ctxbuf.at[r0:r0 + rows, :]

        barrier = pltpu.get_barrier_semaphore()

        @pl.when((my == 1) | (my == 3))
        def _():
            for d in (0, 2):
                pl.semaphore_signal(barrier, inc=1, device_id=(d,),
                                    device_id_type=MESH)
            pl.semaphore_wait(barrier, 1)

        @pl.when(my == 2)
        def _():
            for d in (0, 1, 3):
                pl.semaphore_signal(barrier, inc=1, device_id=(d,),
                                    device_id_type=MESH)
            pl.semaphore_wait(barrier, 2)

        @pl.when(my == 0)
        def _():
            pl.semaphore_wait(barrier, 3)

        @pl.when(my == 0)
        def _producer():
            qbf[...] = (jnp.dot(x_ref[...], wq_ref[...],
                                preferred_element_type=f32)
                        * SCALE).astype(bf)
            ones = jnp.ones((SQ, DH), bf)
            for h in range(HQ):
                vaug[:, h * 2 * DH:h * 2 * DH + DH] = \
                    v_ref[:, h * DH:(h + 1) * DH]
                vaug[:, h * 2 * DH + DH:(h + 1) * 2 * DH] = ones
            biases = {}
            for rows in {256, 128}:
                r = lax.broadcasted_iota(jnp.int32, (rows, rows), 0) // BLK
                c = lax.broadcasted_iota(jnp.int32, (rows, rows), 1) // BLK
                biases[rows] = jnp.where(r >= c, f32(0), f32(-1e9)).astype(bf)

            rdmas = []
            for t in range(N_TILES):
                r0, rows = TILES[t]
                for h in range(HQ):
                    hs = slice(h * DH, (h + 1) * DH)
                    vs = slice(h * 2 * DH, (h + 1) * 2 * DH)
                    qt = qbf[r0:r0 + rows, hs]
                    sd = lax.dot_general(
                        qt, k_ref[r0:r0 + rows, hs], NT,
                        preferred_element_type=f32)
                    wd = jnp.exp(sd.astype(bf) + biases[rows])
                    res = lax.dot_general(
                        wd, vaug[r0:r0 + rows, vs], NN,
                        preferred_element_type=f32)
                    if r0 > 0:
                        s1 = lax.dot_general(
                            qt, k_ref[0:r0, hs], NT,
                            preferred_element_type=f32)
                        w1 = jnp.exp(s1.astype(bf))
                        res = res + lax.dot_general(
                            w1, vaug[0:r0, vs], NN,
                            preferred_element_type=f32)
                    ctxbuf[r0:r0 + rows, hs] = \
                        (res[:, 0:DH] / res[:, DH:DH + 1]).astype(bf)
                dests = ((1,) if t % 2 == 0 else (3,)) if t < N_CHAIN \
                    else (1, 3)
                for i, d in enumerate(dests):
                    rd = pltpu.make_async_remote_copy(
                        src_ref=tile_ref(t), dst_ref=tile_ref(t),
                        send_sem=send_sems.at[t, i], recv_sem=recv_sems.at[t],
                        device_id=(d,), device_id_type=MESH)
                    rd.start()
                    rdmas.append(rd)
            out_ref[...] = jnp.dot(ctxbuf[...], wo_ref[...],
                                   preferred_element_type=f32).astype(bf)
            for rd in rdmas:
                rd.wait_send()

        @pl.when(my != 0)
        def _consumer():
            def recv(t):
                rd = pltpu.make_async_remote_copy(
                    src_ref=tile_ref(t), dst_ref=tile_ref(t),
                    send_sem=send_sems.at[t, 0], recv_sem=recv_sems.at[t],
                    device_id=(0,), device_id_type=MESH)
                rd.wait_recv()

            def fwd_descriptor(t, target):
                return pltpu.make_async_remote_copy(
                    src_ref=tile_ref(t), dst_ref=tile_ref(t),
                    send_sem=fwd_sems.at[t], recv_sem=recv_sems.at[t],
                    device_id=(target,), device_id_type=MESH)

            evens = [t for t in range(N_CHAIN) if t % 2 == 0]
            odds = [t for t in range(N_CHAIN) if t % 2 == 1]
            tree = list(range(N_CHAIN, N_TILES))
            tree_fwd = {t: (1 if t % 2 == 0 else 3) for t in tree}
            fwds = []

            for me, chain_tiles in ((1, evens), ((3), odds)):
                mine = my == me

                @pl.when(mine)
                def _first_hop(chain_tiles=chain_tiles, me=me):
                    for t in chain_tiles:
                        recv(t)
                        fwd_descriptor(t, 2).start()
                    for t in tree:
                        recv(t)
                        if tree_fwd[t] == me:
                            fwd_descriptor(t, 2).start()
                for t in chain_tiles:
                    fwds.append((t, mine, 2))
                for t in tree:
                    if tree_fwd[t] == me:
                        fwds.append((t, mine, 2))

            @pl.when(my == 2)
            def _mid_hop():
                for t in range(N_CHAIN):
                    recv(t)
                    fwd_descriptor(t, 3 if t % 2 == 0 else 1).start()
                for t in tree:
                    recv(t)
            for t in range(N_CHAIN):
                fwds.append((t, my == 2, 3 if t % 2 == 0 else 1))

            @pl.when(my == 1)
            def _last_hop_odds():
                for t in odds:
                    recv(t)

            @pl.when(my == 3)
            def _last_hop_evens():
                for t in evens:
                    recv(t)

            out_ref[...] = jnp.dot(ctxbuf[...], wo_ref[...],
                                   preferred_element_type=f32).astype(bf)

            for t, cond, target in fwds:
                @pl.when(cond)
                def _fwd_wait(t=t, target=target):
                    fwd_descriptor(t, target).wait_send()

    out = pl.pallas_call(
        body,
        out_shape=jax.ShapeDtypeStruct((SQ, D), bf),
        in_specs=[pl.BlockSpec(memory_space=pltpu.VMEM)] * 5,
        out_specs=pl.BlockSpec(memory_space=pltpu.VMEM),
        scratch_shapes=[
            pltpu.VMEM((SQ, D), bf),
            pltpu.VMEM((SQ, 2 * D), bf),
            pltpu.VMEM((SQ, D), bf),
            pltpu.SemaphoreType.DMA((N_TILES, 3)),
            pltpu.SemaphoreType.DMA((N_TILES,)),
            pltpu.SemaphoreType.DMA((N_TILES,)),
        ],
        compiler_params=pltpu.CompilerParams(collective_id=0),
    )(x2, Wqb, K2, V2, Wob)

    return out.reshape(1, SQ, D)


# device time: 66462 ns/iter; 2.2252x vs baseline; 1.0632x over previous
import jax
import jax.numpy as jnp
from jax import lax
from jax.experimental import pallas as pl
from jax.experimental.pallas import tpu as pltpu

SQ = 2048
D = 1024
HQ = 8
DH = 128
BLK = 64
SCALE = 0.08838834764831843
TILES = [(0, 256), (256, 256), (512, 256), (768, 256), (1024, 256),
         (1280, 256), (1536, 256), (1792, 128), (1920, 128)]
N_TILES = len(TILES)
N_CHAIN = 5
NT = (((1,), (1,)), ((), ()))
NN = (((1,), (0,)), ((), ()))
MESH = pl.DeviceIdType.MESH


def kernel(x, Wq, K_ext, V_ext, Wo):
    bf = jnp.bfloat16
    f32 = jnp.float32
    x2 = x.reshape(SQ, D).astype(bf)
    K2 = K_ext.reshape(SQ, D).astype(bf)
    V2 = V_ext.reshape(SQ, D).astype(bf)
    Wqb = Wq.astype(bf)
    Wob = Wo.astype(bf)

    def body(x_ref, wq_ref, k_ref, v_ref, wo_ref, out_ref,
             qbf, vaug, ctxbuf, send_sems, recv_sems, fwd_sems):
        my = lax.axis_index("i")

        def tile_ref(t):
            r0, rows = TILES[t]
            return ctxbuf.at[r0:r0 + rows, :]

        barrier = pltpu.get_barrier_semaphore()

        @pl.when((my == 1) | (my == 3))
        def _():
            for d in (0, 2):
                pl.semaphore_signal(barrier, inc=1, device_id=(d,),
                                    device_id_type=MESH)
            pl.semaphore_wait(barrier, 1)

        @pl.when(my == 2)
        def _():
            for d in (0, 1, 3):
                pl.semaphore_signal(barrier, inc=1, device_id=(d,),
                                    device_id_type=MESH)
            pl.semaphore_wait(barrier, 2)

        @pl.when(my == 0)
        def _():
            pl.semaphore_wait(barrier, 3)

        @pl.when(my == 0)
        def _producer():
            qbf[...] = (jnp.dot(x_ref[...], wq_ref[...],
                                preferred_element_type=f32)
                        * SCALE).astype(bf)
            ones = jnp.ones((SQ, DH), bf)
            for h in range(HQ):
                vaug[:, h * 2 * DH:h * 2 * DH + DH] = \
                    v_ref[:, h * DH:(h + 1) * DH]
                vaug[:, h * 2 * DH + DH:(h + 1) * 2 * DH] = ones
            biases = {}
            for rows in {256, 128}:
                r = lax.broadcasted_iota(jnp.int32, (rows, rows), 0) // BLK
                c = lax.broadcasted_iota(jnp.int32, (rows, rows), 1) // BLK
                biases[rows] = jnp.where(r >= c, f32(0), f32(-1e9)).astype(bf)

            rdmas = []
            for t in range(N_TILES):
                r0, rows = TILES[t]
                for h in range(HQ):
                    hs = slice(h * DH, (h + 1) * DH)
                    vs = slice(h * 2 * DH, (h + 1) * 2 * DH)
                    qt = qbf[r0:r0 + rows, hs]
                    sd = lax.dot_general(
                        qt, k_ref[r0:r0 + rows, hs], NT,
                        preferred_element_type=f32)
                    wd = jnp.exp(sd.astype(bf) + biases[rows])
                    res = lax.dot_general(
                        wd, vaug[r0:r0 + rows, vs], NN,
                        preferred_element_type=f32)
                    if r0 > 0:
                        s1 = lax.dot_general(
                            qt, k_ref[0:r0, hs], NT,
                            preferred_element_type=f32)
                        w1 = jnp.exp(s1.astype(bf))
                        res = res + lax.dot_general(
                            w1, vaug[0:r0, vs], NN,
                            preferred_element_type=f32)
                    ctxbuf[r0:r0 + rows, hs] = \
                        (res[:, 0:DH] / res[:, DH:DH + 1]).astype(bf)
                dests = ((1,) if t % 2 == 0 else (3,)) if t < N_CHAIN \
                    else (1, 3)
                for i, d in enumerate(dests):
                    rd = pltpu.make_async_remote_copy(
                        src_ref=tile_ref(t), dst_ref=tile_ref(t),
                        send_sem=send_sems.at[t, i], recv_sem=recv_sems.at[t],
                        device_id=(d,), device_id_type=MESH)
                    rd.start()
                    rdmas.append(rd)
            out_ref[...] = jnp.dot(ctxbuf[...], wo_ref[...],
                                   preferred_element_type=f32).astype(bf)
            for rd in rdmas:
                rd.wait_send()

        @pl.when(my != 0)
        def _consumer():
            def recv(t):
                rd = pltpu.make_async_remote_copy(
                    src_ref=tile_ref(t), dst_ref=tile_ref(t),
                    send_sem=send_sems.at[t, 0], recv_sem=recv_sems.at[t],
                    device_id=(0,), device_id_type=MESH)
                rd.wait_recv()

            def project(t):
                r0, rows = TILES[t]
                out_ref[r0:r0 + rows, :] = jnp.dot(
                    ctxbuf[r0:r0 + rows, :], wo_ref[...],
                    preferred_element_type=f32).astype(bf)

            def fwd_descriptor(t, target):
                return pltpu.make_async_remote_copy(
                    src_ref=tile_ref(t), dst_ref=tile_ref(t),
                    send_sem=fwd_sems.at[t], recv_sem=recv_sems.at[t],
                    device_id=(target,), device_id_type=MESH)

            evens = [t for t in range(N_CHAIN) if t % 2 == 0]
            odds = [t for t in range(N_CHAIN) if t % 2 == 1]
            tree = list(range(N_CHAIN, N_TILES))
            tree_fwd = {t: (1 if t % 2 == 0 else 3) for t in tree}
            fwds = []

            for me, chain_tiles in ((1, evens), ((3), odds)):
                mine = my == me

                @pl.when(mine)
                def _first_hop(chain_tiles=chain_tiles, me=me):
                    for t in chain_tiles:
                        recv(t)
                        fwd_descriptor(t, 2).start()
                        project(t)
                    for t in tree:
                        recv(t)
                        if tree_fwd[t] == me:
                            fwd_descriptor(t, 2).start()
                        project(t)
                for t in chain_tiles:
                    fwds.append((t, mine, 2))
                for t in tree:
                    if tree_fwd[t] == me:
                        fwds.append((t, mine, 2))

            @pl.when(my == 2)
            def _mid_hop():
                for t in range(N_CHAIN):
                    recv(t)
                    fwd_descriptor(t, 3 if t % 2 == 0 else 1).start()
                    project(t)
                for t in tree:
                    recv(t)
                    project(t)
            for t in range(N_CHAIN):
                fwds.append((t, my == 2, 3 if t % 2 == 0 else 1))

            @pl.when(my == 1)
            def _last_hop_odds():
                for t in odds:
                    recv(t)
                    project(t)

            @pl.when(my == 3)
            def _last_hop_evens():
                for t in evens:
                    recv(t)
                    project(t)

            for t, cond, target in fwds:
                @pl.when(cond)
                def _fwd_wait(t=t, target=target):
                    fwd_descriptor(t, target).wait_send()

    out = pl.pallas_call(
        body,
        out_shape=jax.ShapeDtypeStruct((SQ, D), bf),
        in_specs=[pl.BlockSpec(memory_space=pltpu.VMEM)] * 5,
        out_specs=pl.BlockSpec(memory_space=pltpu.VMEM),
        scratch_shapes=[
            pltpu.VMEM((SQ, D), bf),
            pltpu.VMEM((SQ, 2 * D), bf),
            pltpu.VMEM((SQ, D), bf),
            pltpu.SemaphoreType.DMA((N_TILES, 3)),
            pltpu.SemaphoreType.DMA((N_TILES,)),
            pltpu.SemaphoreType.DMA((N_TILES,)),
        ],
        compiler_params=pltpu.CompilerParams(collective_id=0),
    )(x2, Wqb, K2, V2, Wob)

    return out.reshape(1, SQ, D)
